# baseline (speedup 1.0000x reference)
import functools

import jax
import jax.numpy as jnp
from jax import lax
from jax.experimental import pallas as pl
from jax.experimental.pallas import tpu as pltpu
from jax.experimental.pallas import tpu_sc as plsc

B = 16384
CTX = 20
D = 64
NC = 2
NS = 16
NW = NC * NS
BPW = B // NW
CB = 32
NCHUNK = BPW // CB


_PERM_DN = lax.GatherDimensionNumbers(
    offset_dims=(), collapsed_slice_dims=(0,), start_index_map=(0,))


def _shuf(v, idx):
    return lax.gather(v, idx[:, None], _PERM_DN, (1,),
                      mode=lax.GatherScatterMode.PROMISE_IN_BOUNDS)


def _hsum(v, perms):
    for p in perms:
        v = v + _shuf(v, p)
    return v


def _hmax(v, perms):
    for p in perms:
        v = jnp.maximum(v, _shuf(v, p))
    return v


def _sc_body(tgt_hbm, ctxT_hbm, in_emb_hbm, out_emb_hbm, a_hbm, z_hbm,
             tgt_idx, ctx_idx, tgt_rows, ctx_rows, a_stage, z_stage,
             sem_t, sem_c):
    wid = lax.axis_index("s") * NC + lax.axis_index("c")
    lane = lax.iota(jnp.int32, 16)
    lane0 = lane == 0
    perms = [lane ^ k for k in (8, 4, 2, 1)]
    neg_inf = jnp.float32(-jnp.inf)

    @pl.loop(0, NCHUNK)
    def _chunk(c):
        base = wid * BPW + c * CB

        pltpu.sync_copy(tgt_hbm.at[pl.ds(base, CB)], tgt_idx)
        for w in range(CTX):
            pltpu.sync_copy(ctxT_hbm.at[w, pl.ds(base, CB)], ctx_idx.at[w])

        tcopy = pltpu.async_copy(in_emb_hbm.at[tgt_idx], tgt_rows, sem_t)
        ccopies = [
            pltpu.async_copy(out_emb_hbm.at[ctx_idx.at[w]], ctx_rows.at[w],
                             sem_c)
            for w in range(CTX)
        ]
        tcopy.wait()
        for cc in ccopies:
            cc.wait()

        @pl.loop(0, CB)
        def _row(b):
            t0 = tgt_rows[b, pl.ds(0, 16)]
            t1 = tgt_rows[b, pl.ds(16, 16)]
            t2 = tgt_rows[b, pl.ds(32, 16)]
            t3 = tgt_rows[b, pl.ds(48, 16)]
            s1_vec = None
            sv0 = jnp.full((16,), neg_inf, jnp.float32)
            sv1 = jnp.full((16,), neg_inf, jnp.float32)
            for w in range(CTX):
                c0 = ctx_rows[w, b, pl.ds(0, 16)]
                c1 = ctx_rows[w, b, pl.ds(16, 16)]
                c2 = ctx_rows[w, b, pl.ds(32, 16)]
                c3 = ctx_rows[w, b, pl.ds(48, 16)]
                p = (t0 * c0 + t1 * c1) + (t2 * c2 + t3 * c3)
                s = _hsum(p, perms)
                if w == 1:
                    s1_vec = s
                if w < 16:
                    sv0 = jnp.where(lane == w, s, sv0)
                else:
                    sv1 = jnp.where(lane == (w - 16), s, sv1)
            m = _hmax(jnp.maximum(sv0, sv1), perms)
            z = _hsum(jnp.exp(sv0 - m) + jnp.exp(sv1 - m), perms)
            idxv = jnp.full((16,), b, jnp.int32)
            plsc.store_scatter(a_stage, [idxv], m - s1_vec, mask=lane0)
            plsc.store_scatter(z_stage, [idxv], z, mask=lane0)

        pltpu.sync_copy(a_stage, a_hbm.at[pl.ds(base, CB)])
        pltpu.sync_copy(z_stage, z_hbm.at[pl.ds(base, CB)])


def _finish_body(a_ref, z_ref, o_ref):
    o_ref[0, 0] = jnp.sum(a_ref[...] + jnp.log(z_ref[...])) * (1.0 / B)


@jax.jit
def kernel(target, context, in_embed, out_embed):
    target = target.astype(jnp.int32)
    ctx_t = context.astype(jnp.int32).T

    mesh = plsc.VectorSubcoreMesh(core_axis_name="c", subcore_axis_name="s")
    a, z = pl.kernel(
        _sc_body,
        out_type=(
            jax.ShapeDtypeStruct((B,), jnp.float32),
            jax.ShapeDtypeStruct((B,), jnp.float32),
        ),
        mesh=mesh,
        compiler_params=pltpu.CompilerParams(
            needs_layout_passes=False, use_tc_tiling_on_sc=False),
        scratch_types=[
            pltpu.VMEM((CB,), jnp.int32),
            pltpu.VMEM((CTX, CB), jnp.int32),
            pltpu.VMEM((CB, D), jnp.float32),
            pltpu.VMEM((CTX, CB, D), jnp.float32),
            pltpu.VMEM((CB,), jnp.float32),
            pltpu.VMEM((CB,), jnp.float32),
            pltpu.SemaphoreType.DMA,
            pltpu.SemaphoreType.DMA,
        ],
    )(target, ctx_t, in_embed, out_embed)

    loss = pl.pallas_call(
        _finish_body,
        out_shape=jax.ShapeDtypeStruct((1, 1), jnp.float32),
        out_specs=pl.BlockSpec(memory_space=pltpu.SMEM),
    )(a.reshape(128, 128), z.reshape(128, 128))
    return loss[0, 0]
